# Initial kernel scaffold; baseline (speedup 1.0000x reference)
import jax
import jax.numpy as jnp
from jax.experimental import pallas as pl


def kernel(x, edge_index, i, W1a, W1s, b1, W2a, W2s, b2, W3a, W3s, b3, Wg, a_src, a_dst, bg, Wd, bd):
    raise NotImplementedError("write your pallas kernel here")


# speedup vs baseline: 23.8399x; 23.8399x over previous
import functools
import jax
import jax.numpy as jnp
from jax import lax
from jax.experimental import pallas as pl
from jax.experimental.pallas import tpu as pltpu
from jax.experimental.pallas import tpu_sc as plsc

N = 10000
E = 320000
G = 64
NC = 2
NS = 16
NW = NC * NS
EPT = E // NW
CH = 80
NCHUNK = EPT // CH
NPAD = 10240
RPT = NPAD // NS

_mesh = lambda: plsc.VectorSubcoreMesh(core_axis_name="c", subcore_axis_name="s")

_HIGH = jax.lax.Precision.HIGHEST


def _dot(a, b):
  return jax.lax.dot(a, b, precision=_HIGH, preferred_element_type=jnp.float32)


def _degree_body(src_r, dst_r, out_do, out_di, srcbuf, dstbuf, do_ref, di_ref):
  cid = lax.axis_index("c")
  sid = lax.axis_index("s")
  wid = cid * NS + sid
  pltpu.sync_copy(src_r.at[wid], srcbuf)
  pltpu.sync_copy(dst_r.at[wid], dstbuf)
  zero = jnp.zeros((16,), jnp.float32)

  def zbody(j, _):
    do_ref[pl.ds(j * 16, 16)] = zero
    di_ref[pl.ds(j * 16, 16)] = zero
    return 0
  lax.fori_loop(0, N // 16, zbody, 0)

  ones = jnp.ones((16,), jnp.float32)

  def ebody(j, _):
    for k in range(CH // 16):
      sidx = srcbuf[j, pl.ds(k * 16, 16)]
      didx = dstbuf[j, pl.ds(k * 16, 16)]
      plsc.addupdate_scatter(do_ref, [sidx], ones)
      plsc.addupdate_scatter(di_ref, [didx], ones)
    return 0
  lax.fori_loop(0, NCHUNK, ebody, 0)

  pltpu.sync_copy(do_ref, out_do.at[wid])
  pltpu.sync_copy(di_ref, out_di.at[wid])


def _degrees(src_r, dst_r):
  k = pl.kernel(
      _degree_body,
      out_type=[jax.ShapeDtypeStruct((NW, N), jnp.float32),
                jax.ShapeDtypeStruct((NW, N), jnp.float32)],
      mesh=_mesh(),
      compiler_params=pltpu.CompilerParams(needs_layout_passes=False, use_tc_tiling_on_sc=False),
      scratch_types=[pltpu.VMEM((NCHUNK, CH), jnp.int32),
                     pltpu.VMEM((NCHUNK, CH), jnp.int32),
                     pltpu.VMEM((N,), jnp.float32),
                     pltpu.VMEM((N,), jnp.float32)],
  )
  return k(src_r, dst_r)


def _segsum_body(w, xs, src_r, dst_r, zrows, out, srcbuf, dstbuf, rowbuf,
                 acc, sem):
  cid = lax.axis_index("c")
  sid = lax.axis_index("s")
  wid = cid * NS + sid
  pltpu.sync_copy(zrows, acc.at[pl.ds(sid * RPT, RPT)])
  pltpu.sync_copy(src_r.at[wid], srcbuf)
  pltpu.sync_copy(dst_r.at[wid], dstbuf)
  plsc.subcore_barrier()

  def body(j, _):
    pltpu.async_copy(xs.at[srcbuf.at[j]], rowbuf, sem).wait()
    pltpu.sync_copy(rowbuf, acc.at[dstbuf.at[j]], add=True)
    return 0
  lax.fori_loop(0, NCHUNK, body, 0)

  plsc.subcore_barrier()
  pltpu.sync_copy(acc.at[pl.ds(sid * RPT, RPT)],
                  out.at[cid, pl.ds(sid * RPT, RPT)])


def _segsum(xs, src_r, dst_r, zrows, w):
  k = pl.kernel(
      functools.partial(_segsum_body, w),
      out_type=jax.ShapeDtypeStruct((NC, NPAD, w), jnp.float32),
      mesh=_mesh(),
      compiler_params=pltpu.CompilerParams(needs_layout_passes=False, use_tc_tiling_on_sc=False),
      scratch_types=[pltpu.VMEM((NCHUNK, CH), jnp.int32),
                     pltpu.VMEM((NCHUNK, CH), jnp.int32),
                     pltpu.VMEM((CH, w), jnp.float32),
                     pltpu.VMEM_SHARED((NPAD, w), jnp.float32),
                     pltpu.SemaphoreType.DMA],
  )
  return k(xs, src_r, dst_r, zrows)


def _gat_body(hw, asrc, adst, ma, md, src_r, dst_r, z16, out_num, out_den,
              srcbuf, dstbuf, rowbuf, asbuf, adbuf, mabuf, mdbuf,
              dbuf, accn, sem):
  cid = lax.axis_index("c")
  sid = lax.axis_index("s")
  wid = cid * NS + sid
  pltpu.sync_copy(z16, accn.at[pl.ds(sid * RPT, RPT)])
  pltpu.sync_copy(src_r.at[wid], srcbuf)
  pltpu.sync_copy(dst_r.at[wid], dstbuf)
  pltpu.sync_copy(asrc, asbuf)
  pltpu.sync_copy(adst, adbuf)
  pltpu.sync_copy(ma.at[0, pl.ds(0, 16)], mabuf)
  pltpu.sync_copy(md.at[0, pl.ds(0, 16)], mdbuf)
  zero = jnp.zeros((16,), jnp.float32)

  def zbody(j, _):
    dbuf[pl.ds(j * 16, 16)] = zero
    return 0
  lax.fori_loop(0, N // 16, zbody, 0)
  plsc.subcore_barrier()

  msum = mabuf[...] + mdbuf[...]
  mvec = jnp.maximum(msum, 0.2 * msum)

  def body(j, _):
    pltpu.async_copy(hw.at[srcbuf.at[j]], rowbuf, sem).wait()
    for k in range(CH // 16):
      sidx = srcbuf[j, pl.ds(k * 16, 16)]
      didx = dstbuf[j, pl.ds(k * 16, 16)]
      a_s = plsc.load_gather(asbuf, [sidx])
      a_d = plsc.load_gather(adbuf, [didx])
      l = a_s + a_d
      l = jnp.maximum(l, 0.2 * l)
      ex = jnp.exp(l - mvec)
      plsc.addupdate_scatter(dbuf, [didx], ex)
      for rr in range(16):
        r = k * 16 + rr
        rowbuf[r, :] = rowbuf[r, :] * ex[rr]
    pltpu.sync_copy(rowbuf, accn.at[dstbuf.at[j]], add=True)
    return 0
  lax.fori_loop(0, NCHUNK, body, 0)

  plsc.subcore_barrier()
  pltpu.sync_copy(accn.at[pl.ds(sid * RPT, RPT)],
                  out_num.at[cid, pl.ds(sid * RPT, RPT)])
  pltpu.sync_copy(dbuf, out_den.at[wid])


def _gat_edges(hw, asrc, adst, ma, md, src_r, dst_r, z16):
  k = pl.kernel(
      _gat_body,
      out_type=[jax.ShapeDtypeStruct((NC, NPAD, 16), jnp.float32),
                jax.ShapeDtypeStruct((NW, N), jnp.float32)],
      mesh=_mesh(),
      compiler_params=pltpu.CompilerParams(needs_layout_passes=False, use_tc_tiling_on_sc=False),
      scratch_types=[pltpu.VMEM((NCHUNK, CH), jnp.int32),
                     pltpu.VMEM((NCHUNK, CH), jnp.int32),
                     pltpu.VMEM((CH, 16), jnp.float32),
                     pltpu.VMEM((N,), jnp.float32),
                     pltpu.VMEM((N,), jnp.float32),
                     pltpu.VMEM((16,), jnp.float32),
                     pltpu.VMEM((16,), jnp.float32),
                     pltpu.VMEM((N,), jnp.float32),
                     pltpu.VMEM_SHARED((NPAD, 16), jnp.float32),
                     pltpu.SemaphoreType.DMA],
  )
  return k(hw, asrc, adst, ma, md, src_r, dst_r, z16)


_B = 1000


def _prep_body(do_ref, di_ref, x_ref, ro_ref, ri_ref, xs_ref):
  do = do_ref[0]
  di = di_ref[0]
  ro = lax.rsqrt(jnp.maximum(jnp.sum(do, axis=0), 1.0))[:, None]
  ri = lax.rsqrt(jnp.maximum(jnp.sum(di, axis=0), 1.0))[:, None]
  ro_ref[...] = ro
  ri_ref[...] = ri
  xs_ref[...] = x_ref[...] * ro


def _tc_prep(do_p, di_p, x):
  return pl.pallas_call(
      _prep_body,
      grid=(N // _B,),
      in_specs=[pl.BlockSpec((1, NW, _B), lambda b: (b, 0, 0)),
                pl.BlockSpec((1, NW, _B), lambda b: (b, 0, 0)),
                pl.BlockSpec((_B, 128), lambda b: (b, 0))],
      out_specs=[pl.BlockSpec((_B, 1), lambda b: (b, 0)),
                 pl.BlockSpec((_B, 1), lambda b: (b, 0)),
                 pl.BlockSpec((_B, 128), lambda b: (b, 0))],
      out_shape=[jax.ShapeDtypeStruct((N, 1), jnp.float32),
                 jax.ShapeDtypeStruct((N, 1), jnp.float32),
                 jax.ShapeDtypeStruct((N, 128), jnp.float32)],
  )(do_p, di_p, x)


def _layer1_body(ap_ref, x_ref, ri_ref, ro_ref, w1a_ref, w1s_ref, b1_ref,
                 w2a_ref, w2s_ref, p2a_ref, p2s_ref):
  ap = ap_ref[...]
  agg = (ap[0] + ap[1]) * ri_ref[...]
  h = jnp.maximum(_dot(agg, w1a_ref[...]) + _dot(x_ref[...], w1s_ref[...])
                  + b1_ref[...], 0.0)
  p2a_ref[...] = _dot(h, w2a_ref[...]) * ro_ref[...]
  p2s_ref[...] = _dot(h, w2s_ref[...])


def _tc_layer1(a1p, x, ri, ro, W1a, W1s, b1, W2a, W2s):
  full = lambda r, c: pl.BlockSpec((r, c), lambda b: (0, 0))
  return pl.pallas_call(
      _layer1_body,
      grid=(N // _B,),
      in_specs=[pl.BlockSpec((NC, _B, 128), lambda b: (0, b, 0)),
                pl.BlockSpec((_B, 128), lambda b: (b, 0)),
                pl.BlockSpec((_B, 1), lambda b: (b, 0)),
                pl.BlockSpec((_B, 1), lambda b: (b, 0)),
                full(128, 256), full(128, 256), full(1, 256),
                full(256, 128), full(256, 128)],
      out_specs=[pl.BlockSpec((_B, 128), lambda b: (b, 0)),
                 pl.BlockSpec((_B, 128), lambda b: (b, 0))],
      out_shape=[jax.ShapeDtypeStruct((N, 128), jnp.float32),
                 jax.ShapeDtypeStruct((N, 128), jnp.float32)],
  )(a1p, x, ri, ro, W1a, W1s, b1, W2a, W2s)


def _layer2_body(ap_ref, p2s_ref, ri_ref, ro_ref, b2_ref, w3a_ref, w3s_ref,
                 p3a_ref, p3s_ref):
  ap = ap_ref[...]
  h = jnp.maximum((ap[0] + ap[1]) * ri_ref[...] + p2s_ref[...] + b2_ref[...],
                  0.0)
  p3a_ref[...] = _dot(h, w3a_ref[...]) * ro_ref[...]
  p3s_ref[...] = _dot(h, w3s_ref[...])


def _tc_layer2(a2p, p2s, ri, ro, b2, W3a, W3s):
  full = lambda r, c: pl.BlockSpec((r, c), lambda b: (0, 0))
  return pl.pallas_call(
      _layer2_body,
      grid=(N // _B,),
      in_specs=[pl.BlockSpec((NC, _B, 128), lambda b: (0, b, 0)),
                pl.BlockSpec((_B, 128), lambda b: (b, 0)),
                pl.BlockSpec((_B, 1), lambda b: (b, 0)),
                pl.BlockSpec((_B, 1), lambda b: (b, 0)),
                full(1, 128), full(128, 32), full(128, 32)],
      out_specs=[pl.BlockSpec((_B, 32), lambda b: (b, 0)),
                 pl.BlockSpec((_B, 32), lambda b: (b, 0))],
      out_shape=[jax.ShapeDtypeStruct((N, 32), jnp.float32),
                 jax.ShapeDtypeStruct((N, 32), jnp.float32)],
  )(a2p, p2s, ri, ro, b2, W3a, W3s)


def _layer3_body(ap_ref, p3s_ref, ri_ref, b3_ref, wg_ref, asw_ref, adw_ref,
                 hw_ref, as_ref, ad_ref, ma_ref, md_ref):
  i = pl.program_id(0)
  ap = ap_ref[...]
  h = jnp.maximum((ap[0] + ap[1]) * ri_ref[...] + p3s_ref[...] + b3_ref[...],
                  0.0)
  hw = _dot(h, wg_ref[...])
  hw_ref[...] = hw
  av = jnp.sum(hw * asw_ref[...], axis=1)[:, None]
  dv = jnp.sum(hw * adw_ref[...], axis=1)[:, None]
  as_ref[...] = av
  ad_ref[...] = dv
  prev_a = jnp.where(i == 0, jnp.float32(-3e38), ma_ref[...])
  prev_d = jnp.where(i == 0, jnp.float32(-3e38), md_ref[...])
  ma_ref[...] = jnp.maximum(prev_a, jnp.max(av))
  md_ref[...] = jnp.maximum(prev_d, jnp.max(dv))


def _tc_layer3(a3p, p3s, ri, b3, Wg, a_src, a_dst):
  full = lambda r, c: pl.BlockSpec((r, c), lambda b: (0, 0))
  return pl.pallas_call(
      _layer3_body,
      grid=(N // _B,),
      in_specs=[pl.BlockSpec((NC, _B, 32), lambda b: (0, b, 0)),
                pl.BlockSpec((_B, 32), lambda b: (b, 0)),
                pl.BlockSpec((_B, 1), lambda b: (b, 0)),
                full(1, 32), full(32, 16), full(1, 16), full(1, 16)],
      out_specs=[pl.BlockSpec((_B, 16), lambda b: (b, 0)),
                 pl.BlockSpec((_B, 1), lambda b: (b, 0)),
                 pl.BlockSpec((_B, 1), lambda b: (b, 0)),
                 full(8, 128), full(8, 128)],
      out_shape=[jax.ShapeDtypeStruct((N, 16), jnp.float32),
                 jax.ShapeDtypeStruct((N, 1), jnp.float32),
                 jax.ShapeDtypeStruct((N, 1), jnp.float32),
                 jax.ShapeDtypeStruct((8, 128), jnp.float32),
                 jax.ShapeDtypeStruct((8, 128), jnp.float32)],
  )(a3p, p3s, ri, b3, Wg, a_src, a_dst)


def _final_body(hw_ref, as_ref, ad_ref, ma_ref, md_ref, nump_ref, denp_ref,
                i_ref, bg_ref, wd_ref, bd_ref, out_ref, pool_ref):
  b = pl.program_id(0)
  s = ma_ref[...] + md_ref[...]
  m = jnp.maximum(s, 0.2 * s)[0:1, 0:1]
  l = as_ref[...] + ad_ref[...]
  l = jnp.maximum(l, 0.2 * l)
  exs = jnp.exp(l - m)
  hw = hw_ref[...]
  np_ = nump_ref[...]
  num = np_[0] + np_[1] + exs * hw
  den = jnp.sum(denp_ref[0], axis=0)[:, None] + exs
  gat = num / jnp.maximum(den, 1e-30) + bg_ref[...]
  ids = i_ref[0]
  P = (lax.broadcasted_iota(jnp.int32, (G, _B), 0)
       == jnp.broadcast_to(ids, (G, _B))).astype(jnp.float32)

  @pl.when(b == 0)
  def _():
    pool_ref[...] = jnp.zeros((G, 16), jnp.float32)

  pool_ref[...] += _dot(P, gat)
  out_ref[...] = jax.nn.sigmoid(_dot(pool_ref[...], wd_ref[...]) + bd_ref[...])


def _tc_final(hw, asv, adv, ma, md, nump, denp, ivec, bg, Wd, bd):
  full = lambda r, c: pl.BlockSpec((r, c), lambda b: (0, 0))
  return pl.pallas_call(
      _final_body,
      grid=(N // _B,),
      in_specs=[pl.BlockSpec((_B, 16), lambda b: (b, 0)),
                pl.BlockSpec((_B, 1), lambda b: (b, 0)),
                pl.BlockSpec((_B, 1), lambda b: (b, 0)),
                full(8, 128), full(8, 128),
                pl.BlockSpec((NC, _B, 16), lambda b: (0, b, 0)),
                pl.BlockSpec((1, NW, _B), lambda b: (b, 0, 0)),
                pl.BlockSpec((1, 1, _B), lambda b: (b, 0, 0)),
                full(1, 16), full(16, 16), full(1, 16)],
      out_specs=full(G, 16),
      out_shape=jax.ShapeDtypeStruct((G, 16), jnp.float32),
      scratch_shapes=[pltpu.VMEM((G, 16), jnp.float32)],
  )(hw, asv, adv, ma, md, nump, denp, ivec, bg, Wd, bd)


@jax.jit
def kernel(x, edge_index, i, W1a, W1s, b1, W2a, W2s, b2, W3a, W3s, b3,
           Wg, a_src, a_dst, bg, Wd, bd):
  src_r = edge_index[0].reshape(NW, NCHUNK, CH)
  dst_r = edge_index[1].reshape(NW, NCHUNK, CH)
  z128 = jnp.zeros((RPT, 128), jnp.float32)
  z32 = jnp.zeros((RPT, 32), jnp.float32)
  z16 = jnp.zeros((RPT, 16), jnp.float32)
  ivec = i.astype(jnp.int32).reshape(1, N)

  do_p, di_p = _degrees(src_r, dst_r)
  do3 = do_p.reshape(NW, N // _B, _B).swapaxes(0, 1)
  di3 = di_p.reshape(NW, N // _B, _B).swapaxes(0, 1)
  ro, ri, xs1 = _tc_prep(do3, di3, x)

  a1p = _segsum(xs1, src_r, dst_r, z128, 128)
  p2a, p2s = _tc_layer1(a1p, x, ri, ro, W1a, W1s, b1.reshape(1, 256),
                        W2a, W2s)
  a2p = _segsum(p2a, src_r, dst_r, z128, 128)
  p3a, p3s = _tc_layer2(a2p, p2s, ri, ro, b2.reshape(1, 128), W3a, W3s)
  a3p = _segsum(p3a, src_r, dst_r, z32, 32)
  hw, asv, adv, ma, md = _tc_layer3(a3p, p3s, ri, b3.reshape(1, 32), Wg,
                                    a_src.reshape(1, 16),
                                    a_dst.reshape(1, 16))
  nump, denp = _gat_edges(hw, asv.reshape(N), adv.reshape(N), ma, md,
                          src_r, dst_r, z16)
  denp3 = denp.reshape(NW, N // _B, _B).swapaxes(0, 1)
  i3 = ivec.reshape(N // _B, 1, _B)
  out = _tc_final(hw, asv, adv, ma, md, nump, denp3, i3,
                  bg.reshape(1, 16), Wd, bd.reshape(1, 16))
  return out


# speedup vs baseline: 31.4909x; 1.3209x over previous
import functools
import jax
import jax.numpy as jnp
from jax import lax
from jax.experimental import pallas as pl
from jax.experimental.pallas import tpu as pltpu
from jax.experimental.pallas import tpu_sc as plsc

N = 10000
E = 320000
G = 64
NC = 2
NS = 16
NW = NC * NS
EPT = E // NW
CH = 80
NCHUNK = EPT // CH
NPAD = 10240
RPT = NPAD // NS

_mesh = lambda: plsc.VectorSubcoreMesh(core_axis_name="c", subcore_axis_name="s")

def _dot(a, b):
  return jax.lax.dot(a, b, preferred_element_type=jnp.float32)


def _dot_exact(a, b):
  return jax.lax.dot(a, b, precision=jax.lax.Precision.HIGHEST,
                     preferred_element_type=jnp.float32)


def _degree_body(src_r, dst_r, out_do, out_di, srcbuf, dstbuf, do_ref, di_ref):
  cid = lax.axis_index("c")
  sid = lax.axis_index("s")
  wid = cid * NS + sid
  pltpu.sync_copy(src_r.at[wid], srcbuf)
  pltpu.sync_copy(dst_r.at[wid], dstbuf)
  zero = jnp.zeros((16,), jnp.float32)

  def zbody(j, _):
    do_ref[pl.ds(j * 16, 16)] = zero
    di_ref[pl.ds(j * 16, 16)] = zero
    return 0
  lax.fori_loop(0, N // 16, zbody, 0)

  ones = jnp.ones((16,), jnp.float32)

  def ebody(j, _):
    for k in range(CH // 16):
      sidx = srcbuf[j, pl.ds(k * 16, 16)]
      didx = dstbuf[j, pl.ds(k * 16, 16)]
      plsc.addupdate_scatter(do_ref, [sidx], ones)
      plsc.addupdate_scatter(di_ref, [didx], ones)
    return 0
  lax.fori_loop(0, NCHUNK, ebody, 0)

  pltpu.sync_copy(do_ref, out_do.at[wid])
  pltpu.sync_copy(di_ref, out_di.at[wid])


def _degrees(src_r, dst_r):
  k = pl.kernel(
      _degree_body,
      out_type=[jax.ShapeDtypeStruct((NW, N), jnp.float32),
                jax.ShapeDtypeStruct((NW, N), jnp.float32)],
      mesh=_mesh(),
      compiler_params=pltpu.CompilerParams(needs_layout_passes=False, use_tc_tiling_on_sc=False),
      scratch_types=[pltpu.VMEM((NCHUNK, CH), jnp.int32),
                     pltpu.VMEM((NCHUNK, CH), jnp.int32),
                     pltpu.VMEM((N,), jnp.float32),
                     pltpu.VMEM((N,), jnp.float32)],
  )
  return k(src_r, dst_r)


NBUF = 5
NBUF128 = 2


def _pipeline(gather_issue, drain):
  def run(nbuf):
    for b in range(nbuf - 1):
      gather_issue(b, b % nbuf)

    def body(jj, _):
      for b in range(nbuf):
        j = jj * nbuf + b
        drain(j, b)
        nxt = j + nbuf - 1
        nb = (b + nbuf - 1) % nbuf

        @pl.when(nxt < NCHUNK)
        def _():
          gather_issue(nxt, nb)
      return 0
    lax.fori_loop(0, NCHUNK // nbuf, body, 0)
    base = (NCHUNK // nbuf) * nbuf
    for r in range(NCHUNK % nbuf):
      drain(base + r, (base + r) % nbuf)
  return run


def _segsum_body(w, nbuf, xs, src_r, dst_r, zrows, out, srcbuf, dstbuf,
                 rowbuf, acc, *sems):
  cid = lax.axis_index("c")
  sid = lax.axis_index("s")
  wid = cid * NS + sid
  pltpu.sync_copy(zrows, acc.at[pl.ds(sid * RPT, RPT)])
  pltpu.sync_copy(src_r.at[wid], srcbuf)
  pltpu.sync_copy(dst_r.at[wid], dstbuf)
  plsc.subcore_barrier()

  def gissue(j, b):
    pltpu.async_copy(xs.at[srcbuf.at[j]], rowbuf.at[b], sems[b])

  def drain(j, b):
    pltpu.make_async_copy(xs.at[srcbuf.at[j]], rowbuf.at[b], sems[b]).wait()
    pltpu.sync_copy(rowbuf.at[b], acc.at[dstbuf.at[j]], add=True)

  _pipeline(gissue, drain)(nbuf)

  plsc.subcore_barrier()
  pltpu.sync_copy(acc.at[pl.ds(sid * RPT, RPT)],
                  out.at[cid, pl.ds(sid * RPT, RPT)])


def _segsum(xs, src_r, dst_r, zrows, w):
  nbuf = NBUF128 if w == 128 else NBUF
  k = pl.kernel(
      functools.partial(_segsum_body, w, nbuf),
      out_type=jax.ShapeDtypeStruct((NC, NPAD, w), jnp.float32),
      mesh=_mesh(),
      compiler_params=pltpu.CompilerParams(needs_layout_passes=False, use_tc_tiling_on_sc=False),
      scratch_types=[pltpu.VMEM((NCHUNK, CH), jnp.int32),
                     pltpu.VMEM((NCHUNK, CH), jnp.int32),
                     pltpu.VMEM((nbuf, CH, w), jnp.float32),
                     pltpu.VMEM_SHARED((NPAD, w), jnp.float32)]
                    + [pltpu.SemaphoreType.DMA] * nbuf,
  )
  return k(xs, src_r, dst_r, zrows)


def _gat_body(hw, asrc, adst, ma, md, src_r, dst_r, z16, out_num, out_den,
              srcbuf, dstbuf, rowbuf, asbuf, adbuf, mabuf, mdbuf,
              dbuf, accn, *sems):
  cid = lax.axis_index("c")
  sid = lax.axis_index("s")
  wid = cid * NS + sid
  pltpu.sync_copy(z16, accn.at[pl.ds(sid * RPT, RPT)])
  pltpu.sync_copy(src_r.at[wid], srcbuf)
  pltpu.sync_copy(dst_r.at[wid], dstbuf)
  pltpu.sync_copy(asrc, asbuf)
  pltpu.sync_copy(adst, adbuf)
  pltpu.sync_copy(ma.at[0, pl.ds(0, 16)], mabuf)
  pltpu.sync_copy(md.at[0, pl.ds(0, 16)], mdbuf)
  zero = jnp.zeros((16,), jnp.float32)

  def zbody(j, _):
    dbuf[pl.ds(j * 16, 16)] = zero
    return 0
  lax.fori_loop(0, N // 16, zbody, 0)
  plsc.subcore_barrier()

  msum = mabuf[...] + mdbuf[...]
  mvec = jnp.maximum(msum, 0.2 * msum)

  def gissue(j, b):
    pltpu.async_copy(hw.at[srcbuf.at[j]], rowbuf.at[b], sems[b])

  def drain(j, b):
    pltpu.make_async_copy(hw.at[srcbuf.at[j]], rowbuf.at[b], sems[b]).wait()
    for k in range(CH // 16):
      sidx = srcbuf[j, pl.ds(k * 16, 16)]
      didx = dstbuf[j, pl.ds(k * 16, 16)]
      a_s = plsc.load_gather(asbuf, [sidx])
      a_d = plsc.load_gather(adbuf, [didx])
      l = a_s + a_d
      l = jnp.maximum(l, 0.2 * l)
      ex = jnp.exp(l - mvec)
      plsc.addupdate_scatter(dbuf, [didx], ex)
      for rr in range(16):
        r = k * 16 + rr
        rowbuf[b, r, :] = rowbuf[b, r, :] * ex[rr]
    pltpu.sync_copy(rowbuf.at[b], accn.at[dstbuf.at[j]], add=True)

  _pipeline(gissue, drain)(NBUF)

  plsc.subcore_barrier()
  pltpu.sync_copy(accn.at[pl.ds(sid * RPT, RPT)],
                  out_num.at[cid, pl.ds(sid * RPT, RPT)])
  pltpu.sync_copy(dbuf, out_den.at[wid])


def _gat_edges(hw, asrc, adst, ma, md, src_r, dst_r, z16):
  k = pl.kernel(
      _gat_body,
      out_type=[jax.ShapeDtypeStruct((NC, NPAD, 16), jnp.float32),
                jax.ShapeDtypeStruct((NW, N), jnp.float32)],
      mesh=_mesh(),
      compiler_params=pltpu.CompilerParams(needs_layout_passes=False, use_tc_tiling_on_sc=False),
      scratch_types=[pltpu.VMEM((NCHUNK, CH), jnp.int32),
                     pltpu.VMEM((NCHUNK, CH), jnp.int32),
                     pltpu.VMEM((NBUF, CH, 16), jnp.float32),
                     pltpu.VMEM((N,), jnp.float32),
                     pltpu.VMEM((N,), jnp.float32),
                     pltpu.VMEM((16,), jnp.float32),
                     pltpu.VMEM((16,), jnp.float32),
                     pltpu.VMEM((N,), jnp.float32),
                     pltpu.VMEM_SHARED((NPAD, 16), jnp.float32)]
                    + [pltpu.SemaphoreType.DMA] * NBUF,
  )
  return k(hw, asrc, adst, ma, md, src_r, dst_r, z16)


_B = 1000


def _prep_body(do_ref, di_ref, x_ref, ro_ref, ri_ref, xs_ref):
  do = do_ref[0]
  di = di_ref[0]
  ro = lax.rsqrt(jnp.maximum(jnp.sum(do, axis=0), 1.0))[:, None]
  ri = lax.rsqrt(jnp.maximum(jnp.sum(di, axis=0), 1.0))[:, None]
  ro_ref[...] = ro
  ri_ref[...] = ri
  xs_ref[...] = x_ref[...] * ro


def _tc_prep(do_p, di_p, x):
  return pl.pallas_call(
      _prep_body,
      grid=(N // _B,),
      in_specs=[pl.BlockSpec((1, NW, _B), lambda b: (b, 0, 0)),
                pl.BlockSpec((1, NW, _B), lambda b: (b, 0, 0)),
                pl.BlockSpec((_B, 128), lambda b: (b, 0))],
      out_specs=[pl.BlockSpec((_B, 1), lambda b: (b, 0)),
                 pl.BlockSpec((_B, 1), lambda b: (b, 0)),
                 pl.BlockSpec((_B, 128), lambda b: (b, 0))],
      out_shape=[jax.ShapeDtypeStruct((N, 1), jnp.float32),
                 jax.ShapeDtypeStruct((N, 1), jnp.float32),
                 jax.ShapeDtypeStruct((N, 128), jnp.float32)],
  )(do_p, di_p, x)


def _layer1_body(ap_ref, x_ref, ri_ref, ro_ref, w1a_ref, w1s_ref, b1_ref,
                 w2a_ref, w2s_ref, p2a_ref, p2s_ref):
  ap = ap_ref[...]
  agg = (ap[0] + ap[1]) * ri_ref[...]
  h = jnp.maximum(_dot(agg, w1a_ref[...]) + _dot(x_ref[...], w1s_ref[...])
                  + b1_ref[...], 0.0)
  p2a_ref[...] = _dot(h, w2a_ref[...]) * ro_ref[...]
  p2s_ref[...] = _dot(h, w2s_ref[...])


def _tc_layer1(a1p, x, ri, ro, W1a, W1s, b1, W2a, W2s):
  full = lambda r, c: pl.BlockSpec((r, c), lambda b: (0, 0))
  return pl.pallas_call(
      _layer1_body,
      grid=(N // _B,),
      in_specs=[pl.BlockSpec((NC, _B, 128), lambda b: (0, b, 0)),
                pl.BlockSpec((_B, 128), lambda b: (b, 0)),
                pl.BlockSpec((_B, 1), lambda b: (b, 0)),
                pl.BlockSpec((_B, 1), lambda b: (b, 0)),
                full(128, 256), full(128, 256), full(1, 256),
                full(256, 128), full(256, 128)],
      out_specs=[pl.BlockSpec((_B, 128), lambda b: (b, 0)),
                 pl.BlockSpec((_B, 128), lambda b: (b, 0))],
      out_shape=[jax.ShapeDtypeStruct((N, 128), jnp.float32),
                 jax.ShapeDtypeStruct((N, 128), jnp.float32)],
  )(a1p, x, ri, ro, W1a, W1s, b1, W2a, W2s)


def _layer2_body(ap_ref, p2s_ref, ri_ref, ro_ref, b2_ref, w3a_ref, w3s_ref,
                 p3a_ref, p3s_ref):
  ap = ap_ref[...]
  h = jnp.maximum((ap[0] + ap[1]) * ri_ref[...] + p2s_ref[...] + b2_ref[...],
                  0.0)
  p3a_ref[...] = _dot(h, w3a_ref[...]) * ro_ref[...]
  p3s_ref[...] = _dot(h, w3s_ref[...])


def _tc_layer2(a2p, p2s, ri, ro, b2, W3a, W3s):
  full = lambda r, c: pl.BlockSpec((r, c), lambda b: (0, 0))
  return pl.pallas_call(
      _layer2_body,
      grid=(N // _B,),
      in_specs=[pl.BlockSpec((NC, _B, 128), lambda b: (0, b, 0)),
                pl.BlockSpec((_B, 128), lambda b: (b, 0)),
                pl.BlockSpec((_B, 1), lambda b: (b, 0)),
                pl.BlockSpec((_B, 1), lambda b: (b, 0)),
                full(1, 128), full(128, 32), full(128, 32)],
      out_specs=[pl.BlockSpec((_B, 32), lambda b: (b, 0)),
                 pl.BlockSpec((_B, 32), lambda b: (b, 0))],
      out_shape=[jax.ShapeDtypeStruct((N, 32), jnp.float32),
                 jax.ShapeDtypeStruct((N, 32), jnp.float32)],
  )(a2p, p2s, ri, ro, b2, W3a, W3s)


def _layer3_body(ap_ref, p3s_ref, ri_ref, b3_ref, wg_ref, asw_ref, adw_ref,
                 hw_ref, as_ref, ad_ref, ma_ref, md_ref):
  i = pl.program_id(0)
  ap = ap_ref[...]
  h = jnp.maximum((ap[0] + ap[1]) * ri_ref[...] + p3s_ref[...] + b3_ref[...],
                  0.0)
  hw = _dot(h, wg_ref[...])
  hw_ref[...] = hw
  av = jnp.sum(hw * asw_ref[...], axis=1)[:, None]
  dv = jnp.sum(hw * adw_ref[...], axis=1)[:, None]
  as_ref[...] = av
  ad_ref[...] = dv
  prev_a = jnp.where(i == 0, jnp.float32(-3e38), ma_ref[...])
  prev_d = jnp.where(i == 0, jnp.float32(-3e38), md_ref[...])
  ma_ref[...] = jnp.maximum(prev_a, jnp.max(av))
  md_ref[...] = jnp.maximum(prev_d, jnp.max(dv))


def _tc_layer3(a3p, p3s, ri, b3, Wg, a_src, a_dst):
  full = lambda r, c: pl.BlockSpec((r, c), lambda b: (0, 0))
  return pl.pallas_call(
      _layer3_body,
      grid=(N // _B,),
      in_specs=[pl.BlockSpec((NC, _B, 32), lambda b: (0, b, 0)),
                pl.BlockSpec((_B, 32), lambda b: (b, 0)),
                pl.BlockSpec((_B, 1), lambda b: (b, 0)),
                full(1, 32), full(32, 16), full(1, 16), full(1, 16)],
      out_specs=[pl.BlockSpec((_B, 16), lambda b: (b, 0)),
                 pl.BlockSpec((_B, 1), lambda b: (b, 0)),
                 pl.BlockSpec((_B, 1), lambda b: (b, 0)),
                 full(8, 128), full(8, 128)],
      out_shape=[jax.ShapeDtypeStruct((N, 16), jnp.float32),
                 jax.ShapeDtypeStruct((N, 1), jnp.float32),
                 jax.ShapeDtypeStruct((N, 1), jnp.float32),
                 jax.ShapeDtypeStruct((8, 128), jnp.float32),
                 jax.ShapeDtypeStruct((8, 128), jnp.float32)],
  )(a3p, p3s, ri, b3, Wg, a_src, a_dst)


def _final_body(hw_ref, as_ref, ad_ref, ma_ref, md_ref, nump_ref, denp_ref,
                i_ref, bg_ref, wd_ref, bd_ref, out_ref, pool_ref):
  b = pl.program_id(0)
  s = ma_ref[...] + md_ref[...]
  m = jnp.maximum(s, 0.2 * s)[0:1, 0:1]
  l = as_ref[...] + ad_ref[...]
  l = jnp.maximum(l, 0.2 * l)
  exs = jnp.exp(l - m)
  hw = hw_ref[...]
  np_ = nump_ref[...]
  num = np_[0] + np_[1] + exs * hw
  den = jnp.sum(denp_ref[0], axis=0)[:, None] + exs
  gat = num / jnp.maximum(den, 1e-30) + bg_ref[...]
  ids = i_ref[0]
  P = (lax.broadcasted_iota(jnp.int32, (G, _B), 0)
       == jnp.broadcast_to(ids, (G, _B))).astype(jnp.float32)

  @pl.when(b == 0)
  def _():
    pool_ref[...] = jnp.zeros((G, 16), jnp.float32)

  pool_ref[...] += _dot_exact(P, gat)
  out_ref[...] = jax.nn.sigmoid(_dot(pool_ref[...], wd_ref[...]) + bd_ref[...])


def _tc_final(hw, asv, adv, ma, md, nump, denp, ivec, bg, Wd, bd):
  full = lambda r, c: pl.BlockSpec((r, c), lambda b: (0, 0))
  return pl.pallas_call(
      _final_body,
      grid=(N // _B,),
      in_specs=[pl.BlockSpec((_B, 16), lambda b: (b, 0)),
                pl.BlockSpec((_B, 1), lambda b: (b, 0)),
                pl.BlockSpec((_B, 1), lambda b: (b, 0)),
                full(8, 128), full(8, 128),
                pl.BlockSpec((NC, _B, 16), lambda b: (0, b, 0)),
                pl.BlockSpec((1, NW, _B), lambda b: (b, 0, 0)),
                pl.BlockSpec((1, 1, _B), lambda b: (b, 0, 0)),
                full(1, 16), full(16, 16), full(1, 16)],
      out_specs=full(G, 16),
      out_shape=jax.ShapeDtypeStruct((G, 16), jnp.float32),
      scratch_shapes=[pltpu.VMEM((G, 16), jnp.float32)],
  )(hw, asv, adv, ma, md, nump, denp, ivec, bg, Wd, bd)


@jax.jit
def kernel(x, edge_index, i, W1a, W1s, b1, W2a, W2s, b2, W3a, W3s, b3,
           Wg, a_src, a_dst, bg, Wd, bd):
  src_r = edge_index[0].reshape(NW, NCHUNK, CH)
  dst_r = edge_index[1].reshape(NW, NCHUNK, CH)
  z128 = jnp.zeros((RPT, 128), jnp.float32)
  z32 = jnp.zeros((RPT, 32), jnp.float32)
  z16 = jnp.zeros((RPT, 16), jnp.float32)
  ivec = i.astype(jnp.int32).reshape(1, N)

  do_p, di_p = _degrees(src_r, dst_r)
  do3 = do_p.reshape(NW, N // _B, _B).swapaxes(0, 1)
  di3 = di_p.reshape(NW, N // _B, _B).swapaxes(0, 1)
  ro, ri, xs1 = _tc_prep(do3, di3, x)

  a1p = _segsum(xs1, src_r, dst_r, z128, 128)
  p2a, p2s = _tc_layer1(a1p, x, ri, ro, W1a, W1s, b1.reshape(1, 256),
                        W2a, W2s)
  a2p = _segsum(p2a, src_r, dst_r, z128, 128)
  p3a, p3s = _tc_layer2(a2p, p2s, ri, ro, b2.reshape(1, 128), W3a, W3s)
  a3p = _segsum(p3a, src_r, dst_r, z32, 32)
  hw, asv, adv, ma, md = _tc_layer3(a3p, p3s, ri, b3.reshape(1, 32), Wg,
                                    a_src.reshape(1, 16),
                                    a_dst.reshape(1, 16))
  nump, denp = _gat_edges(hw, asv.reshape(N), adv.reshape(N), ma, md,
                          src_r, dst_r, z16)
  denp3 = denp.reshape(NW, N // _B, _B).swapaxes(0, 1)
  i3 = ivec.reshape(N // _B, 1, _B)
  out = _tc_final(hw, asv, adv, ma, md, nump, denp3, i3,
                  bg.reshape(1, 16), Wd, bd.reshape(1, 16))
  return out


# speedup vs baseline: 42.7542x; 1.3577x over previous
import functools
import jax
import jax.numpy as jnp
from jax import lax
from jax.experimental import pallas as pl
from jax.experimental.pallas import tpu as pltpu
from jax.experimental.pallas import tpu_sc as plsc

N = 10000
E = 320000
G = 64
NC = 2
NS = 16
NW = NC * NS
EPT = E // NW
CH = 80
NCHUNK = EPT // CH
NPAD = 10240
RPT = NPAD // NS

_mesh = lambda: plsc.VectorSubcoreMesh(core_axis_name="c", subcore_axis_name="s")

def _dot(a, b):
  return jax.lax.dot(a, b, preferred_element_type=jnp.float32)


def _dot_exact(a, b):
  return jax.lax.dot(a, b, precision=jax.lax.Precision.HIGHEST,
                     preferred_element_type=jnp.float32)


def _degree_body(src_r, dst_r, out_do, out_di, srcbuf, dstbuf, do_ref, di_ref):
  cid = lax.axis_index("c")
  sid = lax.axis_index("s")
  wid = cid * NS + sid
  pltpu.sync_copy(src_r.at[wid], srcbuf)
  pltpu.sync_copy(dst_r.at[wid], dstbuf)
  zero = jnp.zeros((16,), jnp.float32)

  def zbody(j, _):
    do_ref[pl.ds(j * 16, 16)] = zero
    di_ref[pl.ds(j * 16, 16)] = zero
    return 0
  lax.fori_loop(0, N // 16, zbody, 0)

  ones = jnp.ones((16,), jnp.float32)

  def ebody(j, _):
    for k in range(CH // 16):
      sidx = srcbuf[j, pl.ds(k * 16, 16)]
      didx = dstbuf[j, pl.ds(k * 16, 16)]
      plsc.addupdate_scatter(do_ref, [sidx], ones)
      plsc.addupdate_scatter(di_ref, [didx], ones)
    return 0
  lax.fori_loop(0, NCHUNK, ebody, 0)

  pltpu.sync_copy(do_ref, out_do.at[wid])
  pltpu.sync_copy(di_ref, out_di.at[wid])


def _degrees(src_r, dst_r):
  k = pl.kernel(
      _degree_body,
      out_type=[jax.ShapeDtypeStruct((NW, N), jnp.float32),
                jax.ShapeDtypeStruct((NW, N), jnp.float32)],
      mesh=_mesh(),
      compiler_params=pltpu.CompilerParams(needs_layout_passes=False, use_tc_tiling_on_sc=False),
      scratch_types=[pltpu.VMEM((NCHUNK, CH), jnp.int32),
                     pltpu.VMEM((NCHUNK, CH), jnp.int32),
                     pltpu.VMEM((N,), jnp.float32),
                     pltpu.VMEM((N,), jnp.float32)],
  )
  return k(src_r, dst_r)


NBUF = 5
NBUF128 = 2


def _pipeline(gather_issue, drain):
  def run(nbuf):
    for b in range(nbuf - 1):
      gather_issue(b, b % nbuf)

    def body(jj, _):
      for b in range(nbuf):
        j = jj * nbuf + b
        nxt = j + nbuf - 1
        nb = (b + nbuf - 1) % nbuf

        @pl.when(nxt < NCHUNK)
        def _():
          gather_issue(nxt, nb)
        drain(j, b)
      return 0
    lax.fori_loop(0, NCHUNK // nbuf, body, 0)
    base = (NCHUNK // nbuf) * nbuf
    for r in range(NCHUNK % nbuf):
      drain(base + r, (base + r) % nbuf)
  return run


def _segsum_body(w, nbuf, xs, src_r, dst_r, zrows, out, srcbuf, dstbuf,
                 rowbuf, acc, *sems):
  cid = lax.axis_index("c")
  sid = lax.axis_index("s")
  wid = cid * NS + sid
  pltpu.sync_copy(zrows, acc.at[pl.ds(sid * RPT, RPT)])
  pltpu.sync_copy(src_r.at[wid], srcbuf)
  pltpu.sync_copy(dst_r.at[wid], dstbuf)
  plsc.subcore_barrier()

  def gissue(j, b):
    pltpu.async_copy(xs.at[srcbuf.at[j]], rowbuf.at[b], sems[b])

  def drain(j, b):
    pltpu.make_async_copy(xs.at[srcbuf.at[j]], rowbuf.at[b], sems[b]).wait()
    pltpu.sync_copy(rowbuf.at[b], acc.at[dstbuf.at[j]], add=True)

  _pipeline(gissue, drain)(nbuf)

  plsc.subcore_barrier()
  pltpu.sync_copy(acc.at[pl.ds(sid * RPT, RPT)],
                  out.at[cid, pl.ds(sid * RPT, RPT)])


def _segsum(xs, src_r, dst_r, zrows, w):
  nbuf = NBUF128 if w == 128 else NBUF
  k = pl.kernel(
      functools.partial(_segsum_body, w, nbuf),
      out_type=jax.ShapeDtypeStruct((NC, NPAD, w), jnp.float32),
      mesh=_mesh(),
      compiler_params=pltpu.CompilerParams(needs_layout_passes=False, use_tc_tiling_on_sc=False),
      scratch_types=[pltpu.VMEM((NCHUNK, CH), jnp.int32),
                     pltpu.VMEM((NCHUNK, CH), jnp.int32),
                     pltpu.VMEM((nbuf, CH, w), jnp.float32),
                     pltpu.VMEM_SHARED((NPAD, w), jnp.float32)]
                    + [pltpu.SemaphoreType.DMA] * nbuf,
  )
  return k(xs, src_r, dst_r, zrows)


def _gat_body(hw, asrc, adst, ma, md, src_r, dst_r, z16, out_num, out_den,
              srcbuf, dstbuf, rowbuf, asbuf, adbuf, mabuf, mdbuf,
              dbuf, accn, *sems):
  cid = lax.axis_index("c")
  sid = lax.axis_index("s")
  wid = cid * NS + sid
  pltpu.sync_copy(z16, accn.at[pl.ds(sid * RPT, RPT)])
  pltpu.sync_copy(src_r.at[wid], srcbuf)
  pltpu.sync_copy(dst_r.at[wid], dstbuf)
  pltpu.sync_copy(asrc, asbuf)
  pltpu.sync_copy(adst, adbuf)
  pltpu.sync_copy(ma.at[0, pl.ds(0, 16)], mabuf)
  pltpu.sync_copy(md.at[0, pl.ds(0, 16)], mdbuf)
  zero = jnp.zeros((16,), jnp.float32)

  def zbody(j, _):
    dbuf[pl.ds(j * 16, 16)] = zero
    return 0
  lax.fori_loop(0, N // 16, zbody, 0)
  plsc.subcore_barrier()

  msum = mabuf[...] + mdbuf[...]
  mvec = jnp.maximum(msum, 0.2 * msum)

  def gissue(j, b):
    pltpu.async_copy(hw.at[srcbuf.at[j]], rowbuf.at[b], sems[b])

  def drain(j, b):
    pltpu.make_async_copy(hw.at[srcbuf.at[j]], rowbuf.at[b], sems[b]).wait()
    for k in range(CH // 16):
      sidx = srcbuf[j, pl.ds(k * 16, 16)]
      didx = dstbuf[j, pl.ds(k * 16, 16)]
      a_s = plsc.load_gather(asbuf, [sidx])
      a_d = plsc.load_gather(adbuf, [didx])
      l = a_s + a_d
      l = jnp.maximum(l, 0.2 * l)
      ex = jnp.exp(l - mvec)
      plsc.addupdate_scatter(dbuf, [didx], ex)
      for rr in range(16):
        r = k * 16 + rr
        rowbuf[b, r, :] = rowbuf[b, r, :] * ex[rr]
    pltpu.sync_copy(rowbuf.at[b], accn.at[dstbuf.at[j]], add=True)

  _pipeline(gissue, drain)(NBUF)

  plsc.subcore_barrier()
  pltpu.sync_copy(accn.at[pl.ds(sid * RPT, RPT)],
                  out_num.at[cid, pl.ds(sid * RPT, RPT)])
  pltpu.sync_copy(dbuf, out_den.at[wid])


def _gat_edges(hw, asrc, adst, ma, md, src_r, dst_r, z16):
  k = pl.kernel(
      _gat_body,
      out_type=[jax.ShapeDtypeStruct((NC, NPAD, 16), jnp.float32),
                jax.ShapeDtypeStruct((NW, N), jnp.float32)],
      mesh=_mesh(),
      compiler_params=pltpu.CompilerParams(needs_layout_passes=False, use_tc_tiling_on_sc=False),
      scratch_types=[pltpu.VMEM((NCHUNK, CH), jnp.int32),
                     pltpu.VMEM((NCHUNK, CH), jnp.int32),
                     pltpu.VMEM((NBUF, CH, 16), jnp.float32),
                     pltpu.VMEM((N,), jnp.float32),
                     pltpu.VMEM((N,), jnp.float32),
                     pltpu.VMEM((16,), jnp.float32),
                     pltpu.VMEM((16,), jnp.float32),
                     pltpu.VMEM((N,), jnp.float32),
                     pltpu.VMEM_SHARED((NPAD, 16), jnp.float32)]
                    + [pltpu.SemaphoreType.DMA] * NBUF,
  )
  return k(hw, asrc, adst, ma, md, src_r, dst_r, z16)


_B = 1000


def _prep_body(do_ref, di_ref, x_ref, ro_ref, ri_ref, xs_ref):
  do = do_ref[0]
  di = di_ref[0]
  ro = lax.rsqrt(jnp.maximum(jnp.sum(do, axis=0), 1.0))[:, None]
  ri = lax.rsqrt(jnp.maximum(jnp.sum(di, axis=0), 1.0))[:, None]
  ro_ref[...] = ro
  ri_ref[...] = ri
  xs_ref[...] = x_ref[...] * ro


def _tc_prep(do_p, di_p, x):
  return pl.pallas_call(
      _prep_body,
      grid=(N // _B,),
      in_specs=[pl.BlockSpec((1, NW, _B), lambda b: (b, 0, 0)),
                pl.BlockSpec((1, NW, _B), lambda b: (b, 0, 0)),
                pl.BlockSpec((_B, 128), lambda b: (b, 0))],
      out_specs=[pl.BlockSpec((_B, 1), lambda b: (b, 0)),
                 pl.BlockSpec((_B, 1), lambda b: (b, 0)),
                 pl.BlockSpec((_B, 128), lambda b: (b, 0))],
      out_shape=[jax.ShapeDtypeStruct((N, 1), jnp.float32),
                 jax.ShapeDtypeStruct((N, 1), jnp.float32),
                 jax.ShapeDtypeStruct((N, 128), jnp.float32)],
  )(do_p, di_p, x)


def _layer1_body(ap_ref, x_ref, ri_ref, ro_ref, w1a_ref, w1s_ref, b1_ref,
                 w2a_ref, w2s_ref, p2a_ref, p2s_ref):
  ap = ap_ref[...]
  agg = (ap[0] + ap[1]) * ri_ref[...]
  h = jnp.maximum(_dot(agg, w1a_ref[...]) + _dot(x_ref[...], w1s_ref[...])
                  + b1_ref[...], 0.0)
  p2a_ref[...] = _dot(h, w2a_ref[...]) * ro_ref[...]
  p2s_ref[...] = _dot(h, w2s_ref[...])


def _tc_layer1(a1p, x, ri, ro, W1a, W1s, b1, W2a, W2s):
  full = lambda r, c: pl.BlockSpec((r, c), lambda b: (0, 0))
  return pl.pallas_call(
      _layer1_body,
      grid=(N // _B,),
      in_specs=[pl.BlockSpec((NC, _B, 128), lambda b: (0, b, 0)),
                pl.BlockSpec((_B, 128), lambda b: (b, 0)),
                pl.BlockSpec((_B, 1), lambda b: (b, 0)),
                pl.BlockSpec((_B, 1), lambda b: (b, 0)),
                full(128, 256), full(128, 256), full(1, 256),
                full(256, 128), full(256, 128)],
      out_specs=[pl.BlockSpec((_B, 128), lambda b: (b, 0)),
                 pl.BlockSpec((_B, 128), lambda b: (b, 0))],
      out_shape=[jax.ShapeDtypeStruct((N, 128), jnp.float32),
                 jax.ShapeDtypeStruct((N, 128), jnp.float32)],
  )(a1p, x, ri, ro, W1a, W1s, b1, W2a, W2s)


def _layer2_body(ap_ref, p2s_ref, ri_ref, ro_ref, b2_ref, w3a_ref, w3s_ref,
                 p3a_ref, p3s_ref):
  ap = ap_ref[...]
  h = jnp.maximum((ap[0] + ap[1]) * ri_ref[...] + p2s_ref[...] + b2_ref[...],
                  0.0)
  p3a_ref[...] = _dot(h, w3a_ref[...]) * ro_ref[...]
  p3s_ref[...] = _dot(h, w3s_ref[...])


def _tc_layer2(a2p, p2s, ri, ro, b2, W3a, W3s):
  full = lambda r, c: pl.BlockSpec((r, c), lambda b: (0, 0))
  return pl.pallas_call(
      _layer2_body,
      grid=(N // _B,),
      in_specs=[pl.BlockSpec((NC, _B, 128), lambda b: (0, b, 0)),
                pl.BlockSpec((_B, 128), lambda b: (b, 0)),
                pl.BlockSpec((_B, 1), lambda b: (b, 0)),
                pl.BlockSpec((_B, 1), lambda b: (b, 0)),
                full(1, 128), full(128, 32), full(128, 32)],
      out_specs=[pl.BlockSpec((_B, 32), lambda b: (b, 0)),
                 pl.BlockSpec((_B, 32), lambda b: (b, 0))],
      out_shape=[jax.ShapeDtypeStruct((N, 32), jnp.float32),
                 jax.ShapeDtypeStruct((N, 32), jnp.float32)],
  )(a2p, p2s, ri, ro, b2, W3a, W3s)


def _layer3_body(ap_ref, p3s_ref, ri_ref, b3_ref, wg_ref, asw_ref, adw_ref,
                 hw_ref, as_ref, ad_ref, ma_ref, md_ref):
  i = pl.program_id(0)
  ap = ap_ref[...]
  h = jnp.maximum((ap[0] + ap[1]) * ri_ref[...] + p3s_ref[...] + b3_ref[...],
                  0.0)
  hw = _dot(h, wg_ref[...])
  hw_ref[...] = hw
  av = jnp.sum(hw * asw_ref[...], axis=1)[:, None]
  dv = jnp.sum(hw * adw_ref[...], axis=1)[:, None]
  as_ref[...] = av
  ad_ref[...] = dv
  prev_a = jnp.where(i == 0, jnp.float32(-3e38), ma_ref[...])
  prev_d = jnp.where(i == 0, jnp.float32(-3e38), md_ref[...])
  ma_ref[...] = jnp.maximum(prev_a, jnp.max(av))
  md_ref[...] = jnp.maximum(prev_d, jnp.max(dv))


def _tc_layer3(a3p, p3s, ri, b3, Wg, a_src, a_dst):
  full = lambda r, c: pl.BlockSpec((r, c), lambda b: (0, 0))
  return pl.pallas_call(
      _layer3_body,
      grid=(N // _B,),
      in_specs=[pl.BlockSpec((NC, _B, 32), lambda b: (0, b, 0)),
                pl.BlockSpec((_B, 32), lambda b: (b, 0)),
                pl.BlockSpec((_B, 1), lambda b: (b, 0)),
                full(1, 32), full(32, 16), full(1, 16), full(1, 16)],
      out_specs=[pl.BlockSpec((_B, 16), lambda b: (b, 0)),
                 pl.BlockSpec((_B, 1), lambda b: (b, 0)),
                 pl.BlockSpec((_B, 1), lambda b: (b, 0)),
                 full(8, 128), full(8, 128)],
      out_shape=[jax.ShapeDtypeStruct((N, 16), jnp.float32),
                 jax.ShapeDtypeStruct((N, 1), jnp.float32),
                 jax.ShapeDtypeStruct((N, 1), jnp.float32),
                 jax.ShapeDtypeStruct((8, 128), jnp.float32),
                 jax.ShapeDtypeStruct((8, 128), jnp.float32)],
  )(a3p, p3s, ri, b3, Wg, a_src, a_dst)


def _final_body(hw_ref, as_ref, ad_ref, ma_ref, md_ref, nump_ref, denp_ref,
                i_ref, bg_ref, wd_ref, bd_ref, out_ref, pool_ref):
  b = pl.program_id(0)
  s = ma_ref[...] + md_ref[...]
  m = jnp.maximum(s, 0.2 * s)[0:1, 0:1]
  l = as_ref[...] + ad_ref[...]
  l = jnp.maximum(l, 0.2 * l)
  exs = jnp.exp(l - m)
  hw = hw_ref[...]
  np_ = nump_ref[...]
  num = np_[0] + np_[1] + exs * hw
  den = jnp.sum(denp_ref[0], axis=0)[:, None] + exs
  gat = num / jnp.maximum(den, 1e-30) + bg_ref[...]
  ids = i_ref[0]
  P = (lax.broadcasted_iota(jnp.int32, (G, _B), 0)
       == jnp.broadcast_to(ids, (G, _B))).astype(jnp.float32)

  @pl.when(b == 0)
  def _():
    pool_ref[...] = jnp.zeros((G, 16), jnp.float32)

  pool_ref[...] += _dot_exact(P, gat)
  out_ref[...] = jax.nn.sigmoid(_dot(pool_ref[...], wd_ref[...]) + bd_ref[...])


def _tc_final(hw, asv, adv, ma, md, nump, denp, ivec, bg, Wd, bd):
  full = lambda r, c: pl.BlockSpec((r, c), lambda b: (0, 0))
  return pl.pallas_call(
      _final_body,
      grid=(N // _B,),
      in_specs=[pl.BlockSpec((_B, 16), lambda b: (b, 0)),
                pl.BlockSpec((_B, 1), lambda b: (b, 0)),
                pl.BlockSpec((_B, 1), lambda b: (b, 0)),
                full(8, 128), full(8, 128),
                pl.BlockSpec((NC, _B, 16), lambda b: (0, b, 0)),
                pl.BlockSpec((1, NW, _B), lambda b: (b, 0, 0)),
                pl.BlockSpec((1, 1, _B), lambda b: (b, 0, 0)),
                full(1, 16), full(16, 16), full(1, 16)],
      out_specs=full(G, 16),
      out_shape=jax.ShapeDtypeStruct((G, 16), jnp.float32),
      scratch_shapes=[pltpu.VMEM((G, 16), jnp.float32)],
  )(hw, asv, adv, ma, md, nump, denp, ivec, bg, Wd, bd)


@jax.jit
def kernel(x, edge_index, i, W1a, W1s, b1, W2a, W2s, b2, W3a, W3s, b3,
           Wg, a_src, a_dst, bg, Wd, bd):
  src_r = edge_index[0].reshape(NW, NCHUNK, CH)
  dst_r = edge_index[1].reshape(NW, NCHUNK, CH)
  z128 = jnp.zeros((RPT, 128), jnp.float32)
  z32 = jnp.zeros((RPT, 32), jnp.float32)
  z16 = jnp.zeros((RPT, 16), jnp.float32)
  ivec = i.astype(jnp.int32).reshape(1, N)

  do_p, di_p = _degrees(src_r, dst_r)
  do3 = do_p.reshape(NW, N // _B, _B).swapaxes(0, 1)
  di3 = di_p.reshape(NW, N // _B, _B).swapaxes(0, 1)
  ro, ri, xs1 = _tc_prep(do3, di3, x)

  a1p = _segsum(xs1, src_r, dst_r, z128, 128)
  p2a, p2s = _tc_layer1(a1p, x, ri, ro, W1a, W1s, b1.reshape(1, 256),
                        W2a, W2s)
  a2p = _segsum(p2a, src_r, dst_r, z128, 128)
  p3a, p3s = _tc_layer2(a2p, p2s, ri, ro, b2.reshape(1, 128), W3a, W3s)
  a3p = _segsum(p3a, src_r, dst_r, z32, 32)
  hw, asv, adv, ma, md = _tc_layer3(a3p, p3s, ri, b3.reshape(1, 32), Wg,
                                    a_src.reshape(1, 16),
                                    a_dst.reshape(1, 16))
  nump, denp = _gat_edges(hw, asv.reshape(N), adv.reshape(N), ma, md,
                          src_r, dst_r, z16)
  denp3 = denp.reshape(NW, N // _B, _B).swapaxes(0, 1)
  i3 = ivec.reshape(N // _B, 1, _B)
  out = _tc_final(hw, asv, adv, ma, md, nump, denp3, i3,
                  bg.reshape(1, 16), Wd, bd.reshape(1, 16))
  return out
